# baseline (speedup 1.0000x reference)
import functools

import jax
import jax.numpy as jnp
from jax import lax
from jax.experimental import pallas as pl
from jax.experimental.pallas import tpu as pltpu
from jax.experimental.pallas import tpu_sc as plsc

_E = 64
_C = 2
_NC = 2
_NS = 16
_NW = _NC * _NS
_G = 128
_BLK = 8192


def _project_kernel(x_ref, w_ref, b_ref, out_ref):
    res = lax.dot_general(
        w_ref[...], x_ref[...], (((1,), (1,)), ((), ())),
        preferred_element_type=jnp.float32)
    out_ref[...] = res + b_ref[...]


def _project(table2, Wq, bq):
    rows = table2.shape[0]
    grid = (rows + _BLK - 1) // _BLK
    return pl.pallas_call(
        _project_kernel,
        grid=(grid,),
        in_specs=[
            pl.BlockSpec((_BLK, 2 * _E), lambda i: (i, 0)),
            pl.BlockSpec((8, 2 * _E), lambda i: (0, 0)),
            pl.BlockSpec((8, 1), lambda i: (0, 0)),
        ],
        out_specs=pl.BlockSpec((8, _BLK), lambda i: (0, i)),
        out_shape=jax.ShapeDtypeStruct((8, grid * _BLK), jnp.float32),
    )(table2, Wq, bq)


def _make_gather(n_tokens):
    tok_per_w = n_tokens // _NW
    n_groups = tok_per_w // _G
    mesh = plsc.VectorSubcoreMesh(core_axis_name="c", subcore_axis_name="s")

    @functools.partial(
        pl.kernel,
        out_type=jax.ShapeDtypeStruct((_C, _NW, tok_per_w), jnp.float32),
        mesh=mesh,
        compiler_params=pltpu.CompilerParams(
            needs_layout_passes=False, use_tc_tiling_on_sc=False),
        scratch_types=[
            pltpu.VMEM((tok_per_w,), jnp.int32),
            pltpu.VMEM((tok_per_w,), jnp.float32),
            pltpu.VMEM((tok_per_w,), jnp.float32),
            pltpu.SemaphoreType.DMA,
        ],
    )
    def k(p0_hbm, p1_hbm, idx_hbm, out_hbm, idx_v, s0_v, s1_v, sem):
        wid = lax.axis_index("s") * _NC + lax.axis_index("c")
        pltpu.sync_copy(idx_hbm.at[wid], idx_v)
        handles = []
        for j in range(n_groups):
            sl = pl.ds(j * _G, _G)
            handles.append(
                pltpu.async_copy(p0_hbm.at[idx_v.at[sl]], s0_v.at[sl], sem))
            handles.append(
                pltpu.async_copy(p1_hbm.at[idx_v.at[sl]], s1_v.at[sl], sem))
        for h in handles:
            h.wait()
        pltpu.sync_copy(s0_v, out_hbm.at[0, wid])
        pltpu.sync_copy(s1_v, out_hbm.at[1, wid])

    return k


@jax.jit
def kernel(sentence1, emb_table, W, b):
    batch, seq = sentence1.shape
    n_tokens = batch * seq
    vocab = emb_table.shape[0]
    table2 = emb_table.reshape(vocab // 2, 2 * _E)
    Wq = jnp.zeros((8, 2 * _E), jnp.float32)
    Wq = Wq.at[0, :_E].set(W[0]).at[1, :_E].set(W[1])
    Wq = Wq.at[2, _E:].set(W[0]).at[3, _E:].set(W[1])
    bq = jnp.zeros((8, 1), jnp.float32).at[:4, 0].set(
        jnp.concatenate([b, b]))
    proj = _project(table2, Wq, bq)
    p0 = jnp.stack([proj[0], proj[2]], axis=-1).reshape(-1)
    p1 = jnp.stack([proj[1], proj[3]], axis=-1).reshape(-1)
    idx = sentence1.reshape(_NW, n_tokens // _NW)
    out = _make_gather(n_tokens)(p0, p1, idx)
    return out.reshape(_C, n_tokens).T.reshape(batch, seq, _C)


# speedup vs baseline: 7.0890x; 7.0890x over previous
import functools

import jax
import jax.numpy as jnp
from jax import lax
from jax.experimental import pallas as pl
from jax.experimental.pallas import tpu as pltpu
from jax.experimental.pallas import tpu_sc as plsc

_E = 64
_C = 2
_NC = 2
_NS = 16
_NW = _NC * _NS
_G = 128
_BLKV = 32768


def _project_kernel(x_ref, w_ref, b_ref, out_ref):
    res = lax.dot_general(
        w_ref[...], x_ref[...], (((1,), (0,)), ((), ())),
        preferred_element_type=jnp.float32)
    out_ref[...] = res + b_ref[...]


def _project(tt, Wq, bq):
    vocab = tt.shape[1]
    grid = (vocab + _BLKV - 1) // _BLKV
    return pl.pallas_call(
        _project_kernel,
        grid=(grid,),
        in_specs=[
            pl.BlockSpec((_E, _BLKV), lambda i: (0, i)),
            pl.BlockSpec((8, _E), lambda i: (0, 0)),
            pl.BlockSpec((8, 1), lambda i: (0, 0)),
        ],
        out_specs=pl.BlockSpec((8, _BLKV), lambda i: (0, i)),
        out_shape=jax.ShapeDtypeStruct((8, vocab), jnp.float32),
    )(tt, Wq, bq)


def _make_gather(n_tokens):
    tok_per_w = n_tokens // _NW
    n_groups = tok_per_w // _G
    mesh = plsc.VectorSubcoreMesh(core_axis_name="c", subcore_axis_name="s")

    @functools.partial(
        pl.kernel,
        out_type=jax.ShapeDtypeStruct((_C, _NW, tok_per_w), jnp.float32),
        mesh=mesh,
        compiler_params=pltpu.CompilerParams(
            needs_layout_passes=False, use_tc_tiling_on_sc=False),
        scratch_types=[
            pltpu.VMEM((tok_per_w,), jnp.int32),
            pltpu.VMEM((tok_per_w,), jnp.float32),
            pltpu.VMEM((tok_per_w,), jnp.float32),
            pltpu.SemaphoreType.DMA,
        ],
    )
    def k(p0_hbm, p1_hbm, idx_hbm, out_hbm, idx_v, s0_v, s1_v, sem):
        wid = lax.axis_index("s") * _NC + lax.axis_index("c")
        pltpu.sync_copy(idx_hbm.at[wid], idx_v)
        handles = []
        for j in range(n_groups):
            sl = pl.ds(j * _G, _G)
            handles.append(
                pltpu.async_copy(p0_hbm.at[idx_v.at[sl]], s0_v.at[sl], sem))
            handles.append(
                pltpu.async_copy(p1_hbm.at[idx_v.at[sl]], s1_v.at[sl], sem))
        for h in handles:
            h.wait()
        pltpu.sync_copy(s0_v, out_hbm.at[0, wid])
        pltpu.sync_copy(s1_v, out_hbm.at[1, wid])

    return k


@jax.jit
def kernel(sentence1, emb_table, W, b):
    batch, seq = sentence1.shape
    n_tokens = batch * seq
    tt = emb_table.T
    Wq = jnp.zeros((8, _E), jnp.float32).at[:_C].set(W)
    bq = jnp.zeros((8, 1), jnp.float32).at[:_C, 0].set(b)
    proj = _project(tt, Wq, bq)
    p0 = proj[0]
    p1 = proj[1]
    idx = sentence1.reshape(_NW, n_tokens // _NW)
    out = _make_gather(n_tokens)(p0, p1, idx)
    return out.reshape(_C, n_tokens).T.reshape(batch, seq, _C)


# speedup vs baseline: 9.9098x; 1.3979x over previous
import functools

import jax
import jax.numpy as jnp
from jax import lax
from jax.experimental import pallas as pl
from jax.experimental.pallas import tpu as pltpu
from jax.experimental.pallas import tpu_sc as plsc

_E = 64
_C = 2
_NC = 2
_NS = 16
_NW = _NC * _NS
_G = 128
_BLKV = 32768


def _project_kernel(x_ref, w_ref, b_ref, out0_ref, out1_ref):
    res = lax.dot_general(
        w_ref[...], x_ref[...], (((1,), (0,)), ((), ())),
        preferred_element_type=jnp.float32)
    res = res + b_ref[...]
    out0_ref[...] = res[0]
    out1_ref[...] = res[1]


def _project(tt, Wq, bq):
    vocab = tt.shape[1]
    grid = (vocab + _BLKV - 1) // _BLKV
    return pl.pallas_call(
        _project_kernel,
        grid=(grid,),
        in_specs=[
            pl.BlockSpec((_E, _BLKV), lambda i: (0, i)),
            pl.BlockSpec((8, _E), lambda i: (0, 0)),
            pl.BlockSpec((8, 1), lambda i: (0, 0)),
        ],
        out_specs=[
            pl.BlockSpec((_BLKV,), lambda i: (i,)),
            pl.BlockSpec((_BLKV,), lambda i: (i,)),
        ],
        out_shape=[
            jax.ShapeDtypeStruct((vocab,), jnp.float32),
            jax.ShapeDtypeStruct((vocab,), jnp.float32),
        ],
    )(tt, Wq, bq)


def _make_gather(n_tokens):
    tok_per_w = n_tokens // _NW
    n_groups = tok_per_w // _G
    mesh = plsc.VectorSubcoreMesh(core_axis_name="c", subcore_axis_name="s")

    @functools.partial(
        pl.kernel,
        out_type=jax.ShapeDtypeStruct((_C, _NW, tok_per_w), jnp.float32),
        mesh=mesh,
        compiler_params=pltpu.CompilerParams(
            needs_layout_passes=False, use_tc_tiling_on_sc=False),
        scratch_types=[
            pltpu.VMEM((tok_per_w,), jnp.int32),
            pltpu.VMEM((tok_per_w,), jnp.float32),
            pltpu.VMEM((tok_per_w,), jnp.float32),
            pltpu.SemaphoreType.DMA,
        ],
    )
    def k(p0_hbm, p1_hbm, idx_hbm, out_hbm, idx_v, s0_v, s1_v, sem):
        wid = lax.axis_index("s") * _NC + lax.axis_index("c")
        pltpu.sync_copy(idx_hbm.at[wid], idx_v)
        handles = []
        for j in range(n_groups):
            sl = pl.ds(j * _G, _G)
            handles.append(
                pltpu.async_copy(p0_hbm.at[idx_v.at[sl]], s0_v.at[sl], sem))
            handles.append(
                pltpu.async_copy(p1_hbm.at[idx_v.at[sl]], s1_v.at[sl], sem))
        for h in handles:
            h.wait()
        pltpu.sync_copy(s0_v, out_hbm.at[0, wid])
        pltpu.sync_copy(s1_v, out_hbm.at[1, wid])

    return k


@jax.jit
def kernel(sentence1, emb_table, W, b):
    batch, seq = sentence1.shape
    n_tokens = batch * seq
    tt = emb_table.T
    Wq = jnp.zeros((8, _E), jnp.float32).at[:_C].set(W)
    bq = jnp.zeros((8, 1), jnp.float32).at[:_C, 0].set(b)
    p0, p1 = _project(tt, Wq, bq)
    idx = sentence1.reshape(_NW, n_tokens // _NW)
    out = _make_gather(n_tokens)(p0, p1, idx)
    return out.reshape(_C, n_tokens).T.reshape(batch, seq, _C)


# speedup vs baseline: 9.9991x; 1.0090x over previous
import functools

import jax
import jax.numpy as jnp
from jax import lax
from jax.experimental import pallas as pl
from jax.experimental.pallas import tpu as pltpu
from jax.experimental.pallas import tpu_sc as plsc

_E = 64
_C = 2
_NC = 2
_NS = 16
_NW = _NC * _NS
_G = 128
_BLKV = 32768


def _project_kernel(x_ref, w_ref, b_ref, out0_ref, out1_ref):
    res = lax.dot_general(
        w_ref[...], x_ref[...], (((1,), (0,)), ((), ())),
        preferred_element_type=jnp.float32)
    res = res + b_ref[...]
    out0_ref[...] = res[0]
    out1_ref[...] = res[1]


def _project(tt, Wq, bq):
    vocab = tt.shape[1]
    grid = (vocab + _BLKV - 1) // _BLKV
    return pl.pallas_call(
        _project_kernel,
        grid=(grid,),
        in_specs=[
            pl.BlockSpec((_E, _BLKV), lambda i: (0, i)),
            pl.BlockSpec((_C, _E), lambda i: (0, 0)),
            pl.BlockSpec((_C, 1), lambda i: (0, 0)),
        ],
        out_specs=[
            pl.BlockSpec((_BLKV,), lambda i: (i,)),
            pl.BlockSpec((_BLKV,), lambda i: (i,)),
        ],
        out_shape=[
            jax.ShapeDtypeStruct((vocab,), jnp.float32),
            jax.ShapeDtypeStruct((vocab,), jnp.float32),
        ],
    )(tt, Wq, bq)


def _make_gather(n_tokens):
    tok_per_w = n_tokens // _NW
    n_groups = tok_per_w // _G
    mesh = plsc.VectorSubcoreMesh(core_axis_name="c", subcore_axis_name="s")

    @functools.partial(
        pl.kernel,
        out_type=jax.ShapeDtypeStruct((_C, _NW, tok_per_w), jnp.float32),
        mesh=mesh,
        compiler_params=pltpu.CompilerParams(
            needs_layout_passes=False, use_tc_tiling_on_sc=False),
        scratch_types=[
            pltpu.VMEM((tok_per_w,), jnp.int32),
            pltpu.VMEM((tok_per_w,), jnp.float32),
            pltpu.VMEM((tok_per_w,), jnp.float32),
            pltpu.SemaphoreType.DMA,
        ],
    )
    def k(p0_hbm, p1_hbm, idx_hbm, out_hbm, idx_v, s0_v, s1_v, sem):
        wid = lax.axis_index("s") * _NC + lax.axis_index("c")
        pltpu.sync_copy(idx_hbm.at[wid], idx_v)
        handles = []
        for j in range(n_groups):
            sl = pl.ds(j * _G, _G)
            handles.append(
                pltpu.async_copy(p0_hbm.at[idx_v.at[sl]], s0_v.at[sl], sem))
            handles.append(
                pltpu.async_copy(p1_hbm.at[idx_v.at[sl]], s1_v.at[sl], sem))
        for h in handles:
            h.wait()
        pltpu.sync_copy(s0_v, out_hbm.at[0, wid])
        pltpu.sync_copy(s1_v, out_hbm.at[1, wid])

    return k


@jax.jit
def kernel(sentence1, emb_table, W, b):
    batch, seq = sentence1.shape
    n_tokens = batch * seq
    tt = emb_table.T
    p0, p1 = _project(tt, W, b.reshape(_C, 1))
    idx = sentence1.reshape(_NW, n_tokens // _NW)
    out = _make_gather(n_tokens)(p0, p1, idx)
    return out.reshape(_C, n_tokens).T.reshape(batch, seq, _C)
